# baseline (speedup 1.0000x reference)
import jax
import jax.numpy as jnp
from jax.experimental import pallas as pl
from jax.experimental.pallas import tpu as pltpu

T, B, D_IN, H1, HH, SEG, NP, NK = 512, 16, 17, 300, 256, 32, 36, 16
H1D = H1 // 2
HHD = HH // 2
NSEG = T // SEG
TB = T * B
R = B * NSEG
PAD_P = NP - 1
PAD_K = NK - 1
G = 8

f32 = jnp.float32
bf16 = jnp.bfloat16
_CP = pltpu.CompilerParams(vmem_limit_bytes=100 * 1024 * 1024)


def _dot(a, b):
    return jax.lax.dot_general(a.astype(bf16), b.astype(bf16),
                               (((1,), (0,)), ((), ())),
                               preferred_element_type=f32)


def _dot_t(a, b):
    return jax.lax.dot_general(a.astype(bf16), b.astype(bf16),
                               (((1,), (1,)), ((), ())),
                               preferred_element_type=f32)


def _sig(x):
    return 0.5 + 0.5 * jnp.tanh(0.5 * x)


def _gru_update(gi, gh, h, H):
    r = _sig(gi[:, :H] + gh[:, :H])
    z = _sig(gi[:, H:2 * H] + gh[:, H:2 * H])
    n = jnp.tanh(gi[:, 2 * H:] + r * gh[:, 2 * H:])
    return n + z * (h - n)


def _gru1_kernel(x_ref, wif_ref, wib_ref, whf_ref, whb_ref,
                 bif_ref, bib_ref, bhf_ref, bhb_ref,
                 outf_ref, outb_ref, gif_ref, gib_ref):
    x = x_ref[...]
    gif_ref[...] = _dot(x, wif_ref[...]) + bif_ref[...]
    gib_ref[...] = _dot(x, wib_ref[...]) + bib_ref[...]
    whf = whf_ref[...]
    whb = whb_ref[...]
    bhf = bhf_ref[...]
    bhb = bhb_ref[...]

    def step(t, carry):
        hf, hb = carry
        gf = gif_ref[pl.ds(t * B, B), :]
        hf = _gru_update(gf, _dot(hf, whf) + bhf, hf, H1D)
        outf_ref[pl.ds(t, 1)] = hf[None]
        tb = T - 1 - t
        gb = gib_ref[pl.ds(tb * B, B), :]
        hb = _gru_update(gb, _dot(hb, whb) + bhb, hb, H1D)
        outb_ref[pl.ds(tb, 1)] = hb[None]
        return hf, hb

    h0 = jnp.zeros((B, H1D), f32)
    jax.lax.fori_loop(0, T, step, (h0, h0), unroll=4)


def _hier_kernel(xf_ref, xb_ref, w1f_ref, w2f_ref, w1b_ref, w2b_ref,
                 whf_ref, whb_ref, bif_ref, bib_ref, bhf_ref, bhb_ref,
                 hvf_ref, hvb_ref, gf_ref, gb_ref):
    xf = xf_ref[...]
    xb = xb_ref[...]
    gf_ref[...] = _dot(xf, w1f_ref[...]) + bif_ref[...]
    gf_ref[...] += _dot(xb, w2f_ref[...])
    gb_ref[...] = _dot(xf, w1b_ref[...]) + bib_ref[...]
    gb_ref[...] += _dot(xb, w2b_ref[...])
    whf = whf_ref[...]
    whb = whb_ref[...]
    bhf = bhf_ref[...]
    bhb = bhb_ref[...]

    def step(s, carry):
        hf, hb = carry
        g = gf_ref[pl.ds(s * R, R), :]
        hf = _gru_update(g, _dot(hf, whf) + bhf, hf, HHD)
        hvf_ref[pl.ds(s, 1)] = hf[None]
        sb = SEG - 1 - s
        g2 = gb_ref[pl.ds(sb * R, R), :]
        hb = _gru_update(g2, _dot(hb, whb) + bhb, hb, HHD)
        hvb_ref[pl.ds(sb, 1)] = hb[None]
        return hf, hb

    h0 = jnp.zeros((R, HHD), f32)
    jax.lax.fori_loop(0, SEG, step, (h0, h0), unroll=4)


def _attn_loss_kernel(hvt_ref, xf_ref, xb_ref,
                      wq_ref, bq_ref, wv_ref, bv_ref,
                      wp1_ref, wp2_ref, wp3_ref, bp_ref, wk_ref, bk_ref,
                      pit_ref, ks_ref, out_ref, qp_ref, vp_ref, ctx_ref):
    hvt = hvt_ref[...]
    qp_ref[...] = _dot(hvt, wq_ref[...]) + bq_ref[...]
    vp_ref[...] = _dot(hvt, wv_ref[...]) + bv_ref[...]

    n_blk = G * SEG
    ri = jax.lax.broadcasted_iota(jnp.int32, (n_blk, n_blk), 0) // SEG
    ci = jax.lax.broadcasted_iota(jnp.int32, (n_blk, n_blk), 1) // SEG
    bm = ri == ci

    def ablock(i, _):
        q = qp_ref[pl.ds(i * n_blk, n_blk), :]
        v = vp_ref[pl.ds(i * n_blk, n_blk), :]
        h = hvt_ref[pl.ds(i * n_blk, n_blk), :]
        s = _dot_t(q, v)
        s = jnp.where(bm, s, -1e30)
        m = jnp.max(s, axis=1, keepdims=True)
        e = jnp.where(bm, jnp.exp(s - m), 0.0)
        a = e / jnp.sum(e, axis=1, keepdims=True)
        c = _dot(a, h)
        ctx_ref[pl.ds(i * G, G), :] = c.reshape(G, SEG, HH).sum(axis=1)
        return 0

    jax.lax.fori_loop(0, R // G, ablock, 0, unroll=False)

    ctx = ctx_ref[...]

    L = _dot(ctx, wk_ref[...]) + bk_ref[...]
    mk = jnp.max(L, axis=1, keepdims=True)
    lsek = mk + jnp.log(jnp.sum(jnp.exp(L - mk), axis=1, keepdims=True))
    ksv = ks_ref[...]
    kio = jax.lax.broadcasted_iota(jnp.int32, (TB, NK), 1)
    koh = ((kio == ksv) & (ksv != PAD_K)).astype(f32)
    hist = koh.reshape(R, SEG, NK).sum(axis=1)
    kcnt = jnp.sum(koh)
    ce_k = jnp.sum(hist * (lsek - L)) / jnp.maximum(kcnt, 1.0)

    pv = _dot(ctx, wp3_ref[...])
    pb = jnp.broadcast_to(pv.reshape(NSEG, 1, B, NP),
                          (NSEG, SEG, B, NP)).reshape(TB, NP)
    logits = (_dot(xf_ref[...], wp1_ref[...]) + _dot(xb_ref[...], wp2_ref[...])
              + pb + bp_ref[...])
    mp = jnp.max(logits, axis=1, keepdims=True)
    lsep = mp + jnp.log(jnp.sum(jnp.exp(logits - mp), axis=1, keepdims=True))
    ptv = pit_ref[...]
    pio = jax.lax.broadcasted_iota(jnp.int32, (TB, NP), 1)
    poh = ((pio == ptv) & (ptv != PAD_P)).astype(f32)
    pmask = (ptv != PAD_P).astype(f32)
    pcnt = jnp.sum(poh)
    ce_p = (jnp.sum(pmask * lsep) - jnp.sum(poh * logits)) / jnp.maximum(pcnt, 1.0)

    out_ref[...] = jnp.reshape(ce_p + ce_k, (1, 1))


def kernel(sentences, pitches, keysignatures, sentences_len, eoM,
           rnn_wih_f, rnn_whh_f, rnn_bih_f, rnn_bhh_f,
           rnn_wih_b, rnn_whh_b, rnn_bih_b, rnn_bhh_b,
           hier_wih_f, hier_whh_f, hier_bih_f, hier_bhh_f,
           hier_wih_b, hier_whh_b, hier_bih_b, hier_bhh_b,
           wq, bq, wv, bv, wp, bp, wk, bk):
    x2d = sentences.reshape(TB, D_IN)

    outf, outb = pl.pallas_call(
        _gru1_kernel,
        out_shape=[jax.ShapeDtypeStruct((T, B, H1D), f32)] * 2,
        scratch_shapes=[pltpu.VMEM((TB, 3 * H1D), f32)] * 2,
        compiler_params=_CP,
    )(x2d, rnn_wih_f.T.astype(bf16), rnn_wih_b.T.astype(bf16),
      rnn_whh_f.T.astype(bf16), rnn_whh_b.T.astype(bf16),
      (rnn_bih_f)[None], (rnn_bih_b)[None], (rnn_bhh_f)[None], (rnn_bhh_b)[None])

    def seg_rows(o):
        return (o.reshape(NSEG, SEG, B, H1D)
                 .transpose(1, 0, 2, 3)
                 .reshape(SEG * R, H1D))

    hvf, hvb = pl.pallas_call(
        _hier_kernel,
        out_shape=[jax.ShapeDtypeStruct((SEG, R, HHD), f32)] * 2,
        scratch_shapes=[pltpu.VMEM((SEG * R, 3 * HHD), f32)] * 2,
        compiler_params=_CP,
    )(seg_rows(outf), seg_rows(outb),
      hier_wih_f[:, :H1D].T.astype(bf16), hier_wih_f[:, H1D:].T.astype(bf16),
      hier_wih_b[:, :H1D].T.astype(bf16), hier_wih_b[:, H1D:].T.astype(bf16),
      hier_whh_f.T.astype(bf16), hier_whh_b.T.astype(bf16),
      hier_bih_f[None], hier_bih_b[None], hier_bhh_f[None], hier_bhh_b[None])

    hvt = (jnp.concatenate([hvf, hvb], axis=-1)
              .transpose(1, 0, 2).reshape(R * SEG, HH))
    xf2d = outf.reshape(TB, H1D)
    xb2d = outb.reshape(TB, H1D)
    pit = pitches.reshape(TB, 1)
    ks = (keysignatures.reshape(NSEG, SEG, B)
            .transpose(0, 2, 1).reshape(TB, 1))

    loss = pl.pallas_call(
        _attn_loss_kernel,
        out_shape=jax.ShapeDtypeStruct((1, 1), f32),
        scratch_shapes=[pltpu.VMEM((R * SEG, HH), f32)] * 2
                       + [pltpu.VMEM((R, HH), f32)],
        compiler_params=_CP,
    )(hvt, xf2d, xb2d,
      wq.T.astype(bf16), bq[None], wv.T.astype(bf16), bv[None],
      wp[:, :H1D].T.astype(bf16), wp[:, H1D:H1].T.astype(bf16),
      wp[:, H1:].T.astype(bf16), bp[None],
      wk.T.astype(bf16), bk[None], pit, ks)

    return loss.reshape(())


# speedup vs baseline: 1.8199x; 1.8199x over previous
import jax
import jax.numpy as jnp
from jax.experimental import pallas as pl
from jax.experimental.pallas import tpu as pltpu

T, B, D_IN, H1, HH, SEG, NP, NK = 512, 16, 17, 300, 256, 32, 36, 16
H1D = H1 // 2
HHD = HH // 2
NSEG = T // SEG
TB = T * B
R = B * NSEG
PAD_P = NP - 1
PAD_K = NK - 1
G = 8
H1P = 256

f32 = jnp.float32
bf16 = jnp.bfloat16
_CP = pltpu.CompilerParams(vmem_limit_bytes=60 * 1024 * 1024)


def _dot(a, b):
    return jax.lax.dot_general(a.astype(bf16), b.astype(bf16),
                               (((1,), (0,)), ((), ())),
                               preferred_element_type=f32)


def _dot_t(a, b):
    return jax.lax.dot_general(a.astype(bf16), b.astype(bf16),
                               (((1,), (1,)), ((), ())),
                               preferred_element_type=f32)


def _gru_update(gi, gh, h, H):
    r = jax.nn.sigmoid(gi[:, :H] + gh[:, :H])
    z = jax.nn.sigmoid(gi[:, H:2 * H] + gh[:, H:2 * H])
    n = jnp.tanh(gi[:, 2 * H:] + r * gh[:, 2 * H:])
    return n + z * (h - n)


def _gru1_kernel(x_ref, wif_ref, wib_ref, whf_ref, whb_ref,
                 bif_ref, bib_ref, bhf_ref, bhb_ref,
                 outf_ref, outb_ref, gif_ref, gib_ref):
    x = x_ref[...]
    gif_ref[...] = (_dot(x, wif_ref[...]) + bif_ref[...]).astype(bf16)
    gib_ref[...] = (_dot(x, wib_ref[...]) + bib_ref[...]).astype(bf16)
    whf = whf_ref[...]
    whb = whb_ref[...]
    bhf = bhf_ref[...]
    bhb = bhb_ref[...]

    def step(t, carry):
        hf, hb = carry
        gf = gif_ref[pl.ds(t * B, B), :]
        hf = _gru_update(gf, _dot(hf, whf) + bhf, hf, H1P)
        outf_ref[pl.ds(t, 1)] = hf.astype(bf16)[None]
        tb = T - 1 - t
        gb = gib_ref[pl.ds(tb * B, B), :]
        hb = _gru_update(gb, _dot(hb, whb) + bhb, hb, H1P)
        outb_ref[pl.ds(tb, 1)] = hb.astype(bf16)[None]
        return hf, hb

    h0 = jnp.zeros((B, H1P), f32)
    jax.lax.fori_loop(0, T, step, (h0, h0), unroll=4)


def _hier_kernel(xf_ref, xb_ref, w1f_ref, w2f_ref, w1b_ref, w2b_ref,
                 whf_ref, whb_ref, bif_ref, bib_ref, bhf_ref, bhb_ref,
                 hvf_ref, hvb_ref, gf_ref, gb_ref):
    xf = xf_ref[...]
    xb = xb_ref[...]
    gf_ref[...] = (_dot(xf, w1f_ref[...]) + _dot(xb, w2f_ref[...])
                   + bif_ref[...]).astype(bf16)
    gb_ref[...] = (_dot(xf, w1b_ref[...]) + _dot(xb, w2b_ref[...])
                   + bib_ref[...]).astype(bf16)
    whf = whf_ref[...]
    whb = whb_ref[...]
    bhf = bhf_ref[...]
    bhb = bhb_ref[...]

    def step(s, carry):
        hf, hb = carry
        g = gf_ref[pl.ds(s * R, R), :]
        hf = _gru_update(g, _dot(hf, whf) + bhf, hf, HHD)
        hvf_ref[pl.ds(s, 1)] = hf.astype(bf16)[None]
        sb = SEG - 1 - s
        g2 = gb_ref[pl.ds(sb * R, R), :]
        hb = _gru_update(g2, _dot(hb, whb) + bhb, hb, HHD)
        hvb_ref[pl.ds(sb, 1)] = hb.astype(bf16)[None]
        return hf, hb

    h0 = jnp.zeros((R, HHD), f32)
    jax.lax.fori_loop(0, SEG, step, (h0, h0), unroll=4)


def _attn_loss_kernel(hvt_ref, xf_ref, xb_ref,
                      wq_ref, bq_ref, wv_ref, bv_ref,
                      wp1_ref, wp2_ref, wp3_ref, bp_ref, wk_ref, bk_ref,
                      pit_ref, ks_ref, out_ref, qp_ref, vp_ref, ctx_ref):
    hvt = hvt_ref[...]
    qp_ref[...] = (_dot(hvt, wq_ref[...]) + bq_ref[...]).astype(bf16)
    vp_ref[...] = (_dot(hvt, wv_ref[...]) + bv_ref[...]).astype(bf16)

    n_blk = G * SEG
    ri = jax.lax.broadcasted_iota(jnp.int32, (n_blk, n_blk), 0) // SEG
    ci = jax.lax.broadcasted_iota(jnp.int32, (n_blk, n_blk), 1) // SEG
    bm = ri == ci

    def ablock(i, _):
        q = qp_ref[pl.ds(i * n_blk, n_blk), :]
        v = vp_ref[pl.ds(i * n_blk, n_blk), :]
        h = hvt_ref[pl.ds(i * n_blk, n_blk), :]
        s = _dot_t(q, v)
        s = jnp.where(bm, s, -1e30)
        m = jnp.max(s, axis=1, keepdims=True)
        e = jnp.where(bm, jnp.exp(s - m), 0.0)
        a = e / jnp.sum(e, axis=1, keepdims=True)
        c = _dot(a, h)
        ctx_ref[pl.ds(i * G, G), :] = c.reshape(G, SEG, HH).sum(axis=1).astype(bf16)
        return 0

    jax.lax.fori_loop(0, R // G, ablock, 0, unroll=False)

    ctx = ctx_ref[...]

    L = _dot(ctx, wk_ref[...]) + bk_ref[...]
    mk = jnp.max(L, axis=1, keepdims=True)
    lsek = mk + jnp.log(jnp.sum(jnp.exp(L - mk), axis=1, keepdims=True))
    ksv = ks_ref[...]
    kio = jax.lax.broadcasted_iota(jnp.int32, (TB, NK), 1)
    koh = ((kio == ksv) & (ksv != PAD_K)).astype(f32)
    hist = koh.reshape(R, SEG, NK).sum(axis=1)
    kcnt = jnp.sum(koh)
    ce_k = jnp.sum(hist * (lsek - L)) / jnp.maximum(kcnt, 1.0)

    pv = _dot(ctx, wp3_ref[...])
    pb = jnp.broadcast_to(pv.reshape(NSEG, 1, B, NP),
                          (NSEG, SEG, B, NP)).reshape(TB, NP)
    logits = (_dot(xf_ref[...], wp1_ref[...]) + _dot(xb_ref[...], wp2_ref[...])
              + pb + bp_ref[...])
    mp = jnp.max(logits, axis=1, keepdims=True)
    lsep = mp + jnp.log(jnp.sum(jnp.exp(logits - mp), axis=1, keepdims=True))
    ptv = pit_ref[...]
    pio = jax.lax.broadcasted_iota(jnp.int32, (TB, NP), 1)
    poh = ((pio == ptv) & (ptv != PAD_P)).astype(f32)
    pmask = (ptv != PAD_P).astype(f32)
    pcnt = jnp.sum(poh)
    ce_p = (jnp.sum(pmask * lsep) - jnp.sum(poh * logits)) / jnp.maximum(pcnt, 1.0)

    out_ref[...] = jnp.reshape(ce_p + ce_k, (1, 1))


def kernel(sentences, pitches, keysignatures, sentences_len, eoM,
           rnn_wih_f, rnn_whh_f, rnn_bih_f, rnn_bhh_f,
           rnn_wih_b, rnn_whh_b, rnn_bih_b, rnn_bhh_b,
           hier_wih_f, hier_whh_f, hier_bih_f, hier_bhh_f,
           hier_wih_b, hier_whh_b, hier_bih_b, hier_bhh_b,
           wq, bq, wv, bv, wp, bp, wk, bk):
    x2d = sentences.reshape(TB, D_IN)

    def pad_g_cols(w):
        k = w.shape[0]
        z = jnp.zeros((k, H1P - H1D), w.dtype)
        return jnp.concatenate([w[:, :H1D], z, w[:, H1D:2 * H1D], z,
                                w[:, 2 * H1D:], z], axis=1)

    def pad_rows(w):
        return jnp.concatenate(
            [w, jnp.zeros((H1P - H1D, w.shape[1]), w.dtype)], axis=0)

    def pad_b(b):
        z = jnp.zeros((H1P - H1D,), b.dtype)
        return jnp.concatenate([b[:H1D], z, b[H1D:2 * H1D], z,
                                b[2 * H1D:], z])[None]

    outf, outb = pl.pallas_call(
        _gru1_kernel,
        out_shape=[jax.ShapeDtypeStruct((T, B, H1P), bf16)] * 2,
        scratch_shapes=[pltpu.VMEM((TB, 3 * H1P), bf16)] * 2,
        compiler_params=_CP,
    )(x2d.astype(bf16), pad_g_cols(rnn_wih_f.T).astype(bf16), pad_g_cols(rnn_wih_b.T).astype(bf16),
      pad_rows(pad_g_cols(rnn_whh_f.T)).astype(bf16),
      pad_rows(pad_g_cols(rnn_whh_b.T)).astype(bf16),
      pad_b(rnn_bih_f), pad_b(rnn_bih_b), pad_b(rnn_bhh_f), pad_b(rnn_bhh_b))

    def seg_rows(o):
        return (o.reshape(NSEG, SEG, B, H1P)
                 .transpose(1, 0, 2, 3)
                 .reshape(SEG * R, H1P))

    hvf, hvb = pl.pallas_call(
        _hier_kernel,
        out_shape=[jax.ShapeDtypeStruct((SEG, R, HHD), bf16)] * 2,
        scratch_shapes=[pltpu.VMEM((SEG * R, 3 * HHD), bf16)] * 2,
        compiler_params=_CP,
    )(seg_rows(outf), seg_rows(outb),
      pad_rows(hier_wih_f[:, :H1D].T).astype(bf16),
      pad_rows(hier_wih_f[:, H1D:].T).astype(bf16),
      pad_rows(hier_wih_b[:, :H1D].T).astype(bf16),
      pad_rows(hier_wih_b[:, H1D:].T).astype(bf16),
      hier_whh_f.T.astype(bf16), hier_whh_b.T.astype(bf16),
      hier_bih_f[None], hier_bih_b[None], hier_bhh_f[None], hier_bhh_b[None])

    hvt = (jnp.concatenate([hvf, hvb], axis=-1)
              .transpose(1, 0, 2).reshape(R * SEG, HH))
    xf2d = outf.reshape(TB, H1P)
    xb2d = outb.reshape(TB, H1P)
    pit = pitches.reshape(TB, 1)
    ks = (keysignatures.reshape(NSEG, SEG, B)
            .transpose(0, 2, 1).reshape(TB, 1))

    loss = pl.pallas_call(
        _attn_loss_kernel,
        out_shape=jax.ShapeDtypeStruct((1, 1), f32),
        scratch_shapes=[pltpu.VMEM((R * SEG, HH), bf16)] * 2
                       + [pltpu.VMEM((R, HH), bf16)],
        compiler_params=_CP,
    )(hvt, xf2d, xb2d,
      wq.T.astype(bf16), bq[None], wv.T.astype(bf16), bv[None],
      pad_rows(wp[:, :H1D].T).astype(bf16),
      pad_rows(wp[:, H1D:H1].T).astype(bf16),
      wp[:, H1:].T.astype(bf16), bp[None],
      wk.T.astype(bf16), bk[None], pit, ks)

    return loss.reshape(())
